# Initial kernel scaffold; baseline (speedup 1.0000x reference)
import jax
import jax.numpy as jnp
from jax.experimental import pallas as pl


def kernel(inputs):
    raise NotImplementedError("write your pallas kernel here")


# speedup vs baseline: 7.3233x; 7.3233x over previous
import jax
import jax.numpy as jnp
from jax import lax
from jax.experimental import pallas as pl
from jax.experimental.pallas import tpu as pltpu
from jax.experimental.pallas import tpu_sc as plsc

_NC = 2
_NS = 16
_L = 16


def _make_iwt_sc(B, H, W, C4):
    n = C4 // 4
    R = B * H
    NW = _NC * _NS
    assert R % NW == 0
    rows_per_worker = R // NW
    row_in = W * C4
    row_out = 2 * n * W

    mesh = plsc.VectorSubcoreMesh(
        core_axis_name="c", subcore_axis_name="s",
        num_cores=_NC, num_subcores=_NS)

    def body(x_hbm, y_hbm, xin, yev, yod):
        wid = lax.axis_index("s") * _NC + lax.axis_index("c")

        def row_loop(t, carry):
            i = wid * rows_per_worker + t
            pltpu.sync_copy(x_hbm.at[i], xin)

            def w_loop(w, c2):
                o = w * C4
                eo = w * 2 * n
                for j in range(n // _L):
                    x1 = xin[pl.ds(o + j * _L, _L)]
                    x2 = xin[pl.ds(o + n + j * _L, _L)]
                    x3 = xin[pl.ds(o + 2 * n + j * _L, _L)]
                    x4 = xin[pl.ds(o + 3 * n + j * _L, _L)]
                    s12 = x1 + x2
                    d12 = x1 - x2
                    s34 = x3 + x4
                    d34 = x3 - x4
                    yev[pl.ds(eo + j * _L, _L)] = (d12 - d34) * 0.5
                    yev[pl.ds(eo + n + j * _L, _L)] = (s12 - s34) * 0.5
                    yod[pl.ds(eo + j * _L, _L)] = (d12 + d34) * 0.5
                    yod[pl.ds(eo + n + j * _L, _L)] = (s12 + s34) * 0.5
                return c2

            lax.fori_loop(0, W, w_loop, 0)
            pltpu.sync_copy(yev, y_hbm.at[i, 0])
            pltpu.sync_copy(yod, y_hbm.at[i, 1])
            return carry

        lax.fori_loop(0, rows_per_worker, row_loop, 0)

    return pl.kernel(
        body,
        out_type=jax.ShapeDtypeStruct((R, 2, row_out), jnp.float32),
        mesh=mesh,
        scratch_types=[
            pltpu.VMEM((row_in,), jnp.float32),
            pltpu.VMEM((row_out,), jnp.float32),
            pltpu.VMEM((row_out,), jnp.float32),
        ],
    )


def kernel(inputs):
    B, H, W, C4 = inputs.shape
    n = C4 // 4
    x = inputs.reshape(B * H, W * C4)
    y = _make_iwt_sc(B, H, W, C4)(x)
    return y.reshape(B, 2 * H, 2 * W, n)


# speedup vs baseline: 7.7304x; 1.0556x over previous
import jax
import jax.numpy as jnp
from jax import lax
from jax.experimental import pallas as pl
from jax.experimental.pallas import tpu as pltpu
from jax.experimental.pallas import tpu_sc as plsc

_NC = 2
_NS = 16
_L = 16


def _make_iwt_sc(B, H, W, C4):
    n = C4 // 4
    R = B * H
    NW = _NC * _NS
    assert R % NW == 0 and W % 2 == 0 and n % _L == 0
    rows_per_worker = R // NW
    Wh = W // 2
    chunk_in = Wh * C4
    chunk_out = Wh * 2 * n

    mesh = plsc.VectorSubcoreMesh(
        core_axis_name="c", subcore_axis_name="s",
        num_cores=_NC, num_subcores=_NS)

    def butterfly(xin, yev, yod):
        def w_loop(w, c2):
            o = w * C4
            eo = w * 2 * n
            for j in range(n // _L):
                x1 = xin[pl.ds(o + j * _L, _L)]
                x2 = xin[pl.ds(o + n + j * _L, _L)]
                x3 = xin[pl.ds(o + 2 * n + j * _L, _L)]
                x4 = xin[pl.ds(o + 3 * n + j * _L, _L)]
                s12 = x1 + x2
                d12 = x1 - x2
                s34 = x3 + x4
                d34 = x3 - x4
                yev[pl.ds(eo + j * _L, _L)] = (d12 - d34) * 0.5
                yev[pl.ds(eo + n + j * _L, _L)] = (s12 - s34) * 0.5
                yod[pl.ds(eo + j * _L, _L)] = (d12 + d34) * 0.5
                yod[pl.ds(eo + n + j * _L, _L)] = (s12 + s34) * 0.5
            return c2
        lax.fori_loop(0, Wh, w_loop, 0, unroll=2)

    def body(x_hbm, y_hbm,
             xa, xb, eva, evb, oda, odb,
             sia, sib, soa, sob):
        wid = lax.axis_index("s") * _NC + lax.axis_index("c")
        row0 = wid * rows_per_worker

        def in_a(i):
            return pltpu.make_async_copy(x_hbm.at[i, 0], xa, sia)

        def in_b(i):
            return pltpu.make_async_copy(x_hbm.at[i, 1], xb, sib)

        def out_a_ev(i):
            return pltpu.make_async_copy(eva, y_hbm.at[i, 0, 0], soa)

        def out_a_od(i):
            return pltpu.make_async_copy(oda, y_hbm.at[i, 1, 0], soa)

        def out_b_ev(i):
            return pltpu.make_async_copy(evb, y_hbm.at[i, 0, 1], sob)

        def out_b_od(i):
            return pltpu.make_async_copy(odb, y_hbm.at[i, 1, 1], sob)

        in_a(row0).start()

        def row_loop(k, carry):
            i = row0 + k
            in_a(i).wait()
            in_b(i).start()

            @pl.when(k > 0)
            def _():
                out_a_ev(i - 1).wait()
                out_a_od(i - 1).wait()

            butterfly(xa, eva, oda)
            out_a_ev(i).start()
            out_a_od(i).start()

            in_b(i).wait()

            @pl.when(k < rows_per_worker - 1)
            def _():
                in_a(i + 1).start()

            @pl.when(k > 0)
            def _():
                out_b_ev(i - 1).wait()
                out_b_od(i - 1).wait()

            butterfly(xb, evb, odb)
            out_b_ev(i).start()
            out_b_od(i).start()
            return carry

        lax.fori_loop(0, rows_per_worker, row_loop, 0)
        last = row0 + rows_per_worker - 1
        out_a_ev(last).wait()
        out_a_od(last).wait()
        out_b_ev(last).wait()
        out_b_od(last).wait()

    return pl.kernel(
        body,
        out_type=jax.ShapeDtypeStruct((R, 2, 2, chunk_out), jnp.float32),
        mesh=mesh,
        scratch_types=[
            pltpu.VMEM((chunk_in,), jnp.float32),
            pltpu.VMEM((chunk_in,), jnp.float32),
            pltpu.VMEM((chunk_out,), jnp.float32),
            pltpu.VMEM((chunk_out,), jnp.float32),
            pltpu.VMEM((chunk_out,), jnp.float32),
            pltpu.VMEM((chunk_out,), jnp.float32),
            pltpu.SemaphoreType.DMA,
            pltpu.SemaphoreType.DMA,
            pltpu.SemaphoreType.DMA,
            pltpu.SemaphoreType.DMA,
        ],
    )


def kernel(inputs):
    B, H, W, C4 = inputs.shape
    n = C4 // 4
    x = inputs.reshape(B * H, 2, (W // 2) * C4)
    y = _make_iwt_sc(B, H, W, C4)(x)
    return y.reshape(B, 2 * H, 2 * W, n)


# speedup vs baseline: 8.6052x; 1.1132x over previous
import jax
import jax.numpy as jnp
from jax import lax
from jax.experimental import pallas as pl
from jax.experimental.pallas import tpu as pltpu
from jax.experimental.pallas import tpu_sc as plsc

_NC = 2
_NS = 16
_L = 16


def _make_iwt_sc(B, H, W, C4):
    n = C4 // 4
    R = B * H
    NW = _NC * _NS
    assert R % NW == 0 and W % 2 == 0 and n % _L == 0
    rows_per_worker = R // NW
    Wh = W // 2
    chunk_in = Wh * C4
    chunk_out = Wh * 2 * n

    mesh = plsc.VectorSubcoreMesh(
        core_axis_name="c", subcore_axis_name="s",
        num_cores=_NC, num_subcores=_NS)

    def butterfly(xin, yev, yod):
        @plsc.parallel_loop(0, Wh, unroll=4)
        def _(w):
            o = w * C4
            eo = w * 2 * n
            for j in range(n // _L):
                x1 = xin[pl.ds(o + j * _L, _L)]
                x2 = xin[pl.ds(o + n + j * _L, _L)]
                x3 = xin[pl.ds(o + 2 * n + j * _L, _L)]
                x4 = xin[pl.ds(o + 3 * n + j * _L, _L)]
                s12 = x1 + x2
                d12 = x1 - x2
                s34 = x3 + x4
                d34 = x3 - x4
                yev[pl.ds(eo + j * _L, _L)] = (d12 - d34) * 0.5
                yev[pl.ds(eo + n + j * _L, _L)] = (s12 - s34) * 0.5
                yod[pl.ds(eo + j * _L, _L)] = (d12 + d34) * 0.5
                yod[pl.ds(eo + n + j * _L, _L)] = (s12 + s34) * 0.5

    def body(x_hbm, y_hbm,
             xa, xb, eva, evb, oda, odb,
             sia, sib, soa, sob):
        wid = lax.axis_index("s") * _NC + lax.axis_index("c")
        row0 = wid * rows_per_worker

        def in_a(i):
            return pltpu.make_async_copy(x_hbm.at[i, 0], xa, sia)

        def in_b(i):
            return pltpu.make_async_copy(x_hbm.at[i, 1], xb, sib)

        def out_a_ev(i):
            return pltpu.make_async_copy(eva, y_hbm.at[i, 0, 0], soa)

        def out_a_od(i):
            return pltpu.make_async_copy(oda, y_hbm.at[i, 1, 0], soa)

        def out_b_ev(i):
            return pltpu.make_async_copy(evb, y_hbm.at[i, 0, 1], sob)

        def out_b_od(i):
            return pltpu.make_async_copy(odb, y_hbm.at[i, 1, 1], sob)

        in_a(row0).start()

        def row_loop(k, carry):
            i = row0 + k
            in_a(i).wait()
            in_b(i).start()

            @pl.when(k > 0)
            def _():
                out_a_ev(i - 1).wait()
                out_a_od(i - 1).wait()

            butterfly(xa, eva, oda)
            out_a_ev(i).start()
            out_a_od(i).start()

            in_b(i).wait()

            @pl.when(k < rows_per_worker - 1)
            def _():
                in_a(i + 1).start()

            @pl.when(k > 0)
            def _():
                out_b_ev(i - 1).wait()
                out_b_od(i - 1).wait()

            butterfly(xb, evb, odb)
            out_b_ev(i).start()
            out_b_od(i).start()
            return carry

        lax.fori_loop(0, rows_per_worker, row_loop, 0)
        last = row0 + rows_per_worker - 1
        out_a_ev(last).wait()
        out_a_od(last).wait()
        out_b_ev(last).wait()
        out_b_od(last).wait()

    return pl.kernel(
        body,
        out_type=jax.ShapeDtypeStruct((R, 2, 2, chunk_out), jnp.float32),
        mesh=mesh,
        scratch_types=[
            pltpu.VMEM((chunk_in,), jnp.float32),
            pltpu.VMEM((chunk_in,), jnp.float32),
            pltpu.VMEM((chunk_out,), jnp.float32),
            pltpu.VMEM((chunk_out,), jnp.float32),
            pltpu.VMEM((chunk_out,), jnp.float32),
            pltpu.VMEM((chunk_out,), jnp.float32),
            pltpu.SemaphoreType.DMA,
            pltpu.SemaphoreType.DMA,
            pltpu.SemaphoreType.DMA,
            pltpu.SemaphoreType.DMA,
        ],
    )


def kernel(inputs):
    B, H, W, C4 = inputs.shape
    n = C4 // 4
    x = inputs.reshape(B * H, 2, (W // 2) * C4)
    y = _make_iwt_sc(B, H, W, C4)(x)
    return y.reshape(B, 2 * H, 2 * W, n)


# speedup vs baseline: 19.3262x; 2.2459x over previous
import jax
import jax.numpy as jnp
from jax import lax
from jax.experimental import pallas as pl
from jax.experimental.pallas import tpu as pltpu
from jax.experimental.pallas import tpu_sc as plsc

_NC = 2
_NS = 16
_L = 16


def _make_iwt_sc(B, H, W, C4):
    n = C4 // 4
    R = B * H
    NW = _NC * _NS
    assert R % NW == 0 and W % 2 == 0 and n % _L == 0
    rows_per_worker = R // NW
    Wh = W // 2

    mesh = plsc.VectorSubcoreMesh(
        core_axis_name="c", subcore_axis_name="s",
        num_cores=_NC, num_subcores=_NS)

    def butterfly(xin, yev, yod):
        @plsc.parallel_loop(0, Wh, unroll=4)
        def _(w):
            for k in range(n // _L):
                x1 = xin[w, pl.ds(k * _L, _L)]
                x2 = xin[w, pl.ds(n + k * _L, _L)]
                x3 = xin[w, pl.ds(2 * n + k * _L, _L)]
                x4 = xin[w, pl.ds(3 * n + k * _L, _L)]
                s12 = x1 + x2
                d12 = x1 - x2
                s34 = x3 + x4
                d34 = x3 - x4
                yev[2 * w, pl.ds(k * _L, _L)] = (d12 - d34) * 0.5
                yev[2 * w + 1, pl.ds(k * _L, _L)] = (s12 - s34) * 0.5
                yod[2 * w, pl.ds(k * _L, _L)] = (d12 + d34) * 0.5
                yod[2 * w + 1, pl.ds(k * _L, _L)] = (s12 + s34) * 0.5

    def body(x_hbm, y_hbm,
             xa, xb, eva, evb, oda, odb,
             sia, sib, soa, sob):
        wid = lax.axis_index("s") * _NC + lax.axis_index("c")
        row0 = wid * rows_per_worker

        def split(i):
            return i // H, i % H

        def in_cp(i, half, buf, sem):
            b, h = split(i)
            return pltpu.make_async_copy(
                x_hbm.at[b, h, pl.ds(half * Wh, Wh)], buf, sem)

        def out_cp(i, parity, half, buf, sem):
            b, h = split(i)
            return pltpu.make_async_copy(
                buf, y_hbm.at[b, 2 * h + parity, pl.ds(half * W, W)], sem)

        in_cp(row0, 0, xa, sia).start()

        def row_loop(k, carry):
            i = row0 + k
            in_cp(i, 0, xa, sia).wait()
            in_cp(i, 1, xb, sib).start()

            @pl.when(k > 0)
            def _():
                out_cp(i - 1, 0, 0, eva, soa).wait()
                out_cp(i - 1, 1, 0, oda, soa).wait()

            butterfly(xa, eva, oda)
            out_cp(i, 0, 0, eva, soa).start()
            out_cp(i, 1, 0, oda, soa).start()

            in_cp(i, 1, xb, sib).wait()

            @pl.when(k < rows_per_worker - 1)
            def _():
                in_cp(i + 1, 0, xa, sia).start()

            @pl.when(k > 0)
            def _():
                out_cp(i - 1, 0, 1, evb, sob).wait()
                out_cp(i - 1, 1, 1, odb, sob).wait()

            butterfly(xb, evb, odb)
            out_cp(i, 0, 1, evb, sob).start()
            out_cp(i, 1, 1, odb, sob).start()
            return carry

        lax.fori_loop(0, rows_per_worker, row_loop, 0)
        last = row0 + rows_per_worker - 1
        out_cp(last, 0, 0, eva, soa).wait()
        out_cp(last, 1, 0, oda, soa).wait()
        out_cp(last, 0, 1, evb, sob).wait()
        out_cp(last, 1, 1, odb, sob).wait()

    return pl.kernel(
        body,
        out_type=jax.ShapeDtypeStruct((B, 2 * H, 2 * W, n), jnp.float32),
        mesh=mesh,
        scratch_types=[
            pltpu.VMEM((Wh, C4), jnp.float32),
            pltpu.VMEM((Wh, C4), jnp.float32),
            pltpu.VMEM((W, n), jnp.float32),
            pltpu.VMEM((W, n), jnp.float32),
            pltpu.VMEM((W, n), jnp.float32),
            pltpu.VMEM((W, n), jnp.float32),
            pltpu.SemaphoreType.DMA,
            pltpu.SemaphoreType.DMA,
            pltpu.SemaphoreType.DMA,
            pltpu.SemaphoreType.DMA,
        ],
    )


def kernel(inputs):
    B, H, W, C4 = inputs.shape
    return _make_iwt_sc(B, H, W, C4)(inputs)
